# Initial kernel scaffold; baseline (speedup 1.0000x reference)
import jax
import jax.numpy as jnp
from jax.experimental import pallas as pl


def kernel(cls_head_0, reg_head_0, center_head_0, cls_head_1, reg_head_1, center_head_1, cls_head_2, reg_head_2, center_head_2, cls_head_3, reg_head_3, center_head_3, cls_head_4, reg_head_4, center_head_4):
    raise NotImplementedError("write your pallas kernel here")


# speedup vs baseline: 268.4838x; 268.4838x over previous
import functools
import numpy as np
import jax
import jax.numpy as jnp
from jax import lax
from jax.experimental import pallas as pl
from jax.experimental.pallas import tpu as pltpu
from jax.experimental.pallas import tpu_sc as plsc

IMAGE_W = 1024
IMAGE_H = 1024
STRIDES = (8, 16, 32, 64, 128)
TOP_N = 1000
MIN_SCORE = 0.05
NMS_TH = 0.6
MAX_DET = 100
NUM_CLASSES = 80
BATCH = 8

PS = tuple((IMAGE_H // s) ** 2 for s in STRIDES)
NTOT = sum(PS)
NBLK = 171
NPAD = NBLK * 128
NBPAD = 176
BOUNDS = tuple(int(x) for x in np.cumsum((0,) + PS))
CAPS = tuple(min(TOP_N, p) for p in PS)
TOTAL_CAND = sum(CAPS)
OUTP = 112


def _dense_body(stride, f, ch, cls_ref, reg_ref, ctr_ref, sco_ref, pa_ref, pb_ref):
    i = pl.program_id(0)
    sig = jax.nn.sigmoid(cls_ref[...])
    ms = jnp.max(sig, axis=2)
    iot = lax.broadcasted_iota(jnp.int32, (BATCH, ch, NUM_CLASSES), 2)
    argm = jnp.min(jnp.where(sig == ms[:, :, None], iot, NUM_CLASSES), axis=2)
    ctr = jax.nn.sigmoid(ctr_ref[...])
    score = jnp.sqrt(ms * ctr)
    reg = jnp.exp(reg_ref[...])
    p = lax.broadcasted_iota(jnp.int32, (BATCH, ch), 1) + i * ch
    a = p // f
    b = p - a * f
    px = (b.astype(jnp.float32) + 0.5) * stride
    py = (a.astype(jnp.float32) + 0.5) * stride
    x1 = jnp.floor(jnp.maximum(px - reg[:, :, 0], 0.0)).astype(jnp.int32)
    y1 = jnp.floor(jnp.maximum(py - reg[:, :, 1], 0.0)).astype(jnp.int32)
    x2 = jnp.minimum(jnp.floor(px + reg[:, :, 2]), IMAGE_W - 1.0).astype(jnp.int32)
    y2 = jnp.minimum(jnp.floor(py + reg[:, :, 3]), IMAGE_H - 1.0).astype(jnp.int32)
    sco_ref[...] = score
    pa_ref[...] = x1 | (y1 << 10) | (argm << 20)
    pb_ref[...] = x2 | (y2 << 10)


def _dense_call(li):
    s = STRIDES[li]
    f = IMAGE_H // s
    P = PS[li]
    ch = min(512, P)
    grid = P // ch
    return pl.pallas_call(
        functools.partial(_dense_body, float(s), f, ch),
        grid=(grid,),
        in_specs=[
            pl.BlockSpec((BATCH, ch, NUM_CLASSES), lambda i: (0, i, 0)),
            pl.BlockSpec((BATCH, ch, 4), lambda i: (0, i, 0)),
            pl.BlockSpec((BATCH, ch), lambda i: (0, i)),
        ],
        out_specs=[
            pl.BlockSpec((BATCH, ch), lambda i: (0, i)),
            pl.BlockSpec((BATCH, ch), lambda i: (0, i)),
            pl.BlockSpec((BATCH, ch), lambda i: (0, i)),
        ],
        out_shape=[
            jax.ShapeDtypeStruct((BATCH, P), jnp.float32),
            jax.ShapeDtypeStruct((BATCH, P), jnp.int32),
            jax.ShapeDtypeStruct((BATCH, P), jnp.int32),
        ],
    )


def _sc_decode(scores, pa, pb):
    mesh = plsc.VectorSubcoreMesh(core_axis_name="c", subcore_axis_name="s")

    @functools.partial(
        pl.kernel,
        mesh=mesh,
        compiler_params=pltpu.CompilerParams(needs_layout_passes=False),
        out_type=[
            jax.ShapeDtypeStruct((BATCH, OUTP), jnp.float32),
            jax.ShapeDtypeStruct((BATCH, OUTP), jnp.float32),
            jax.ShapeDtypeStruct((BATCH, 4 * OUTP), jnp.float32),
        ],
        scratch_types=[
            pltpu.VMEM((NPAD,), jnp.float32),
            pltpu.VMEM((NPAD,), jnp.int32),
            pltpu.VMEM((NPAD,), jnp.int32),
            pltpu.VMEM((NBPAD,), jnp.float32),
            pltpu.VMEM((OUTP,), jnp.float32),
            pltpu.VMEM((OUTP,), jnp.float32),
            pltpu.VMEM((OUTP,), jnp.float32),
            pltpu.VMEM((OUTP,), jnp.float32),
            pltpu.VMEM((OUTP,), jnp.float32),
            pltpu.VMEM((OUTP,), jnp.float32),
            pltpu.VMEM((OUTP,), jnp.float32),
            pltpu.VMEM((4 * OUTP,), jnp.float32),
        ],
    )
    def k(sco_hbm, pa_hbm, pb_hbm, outs_hbm, outc_hbm, outb_hbm,
          sco_v, pa_v, pb_v, bm_v, kx1_v, ky1_v, kx2_v, ky2_v, kar_v,
          outs_v, outc_v, outb_v):
        wid = lax.axis_index("s") * 2 + lax.axis_index("c")

        @pl.when(wid < BATCH)
        def _():
            img = wid
            pltpu.sync_copy(sco_hbm.at[img], sco_v)
            pltpu.sync_copy(pa_hbm.at[img], pa_v)
            pltpu.sync_copy(pb_hbm.at[img], pb_v)
            iota = lax.iota(jnp.int32, 16)
            neg = jnp.full((16,), -jnp.inf, jnp.float32)
            lane0 = iota == 0

            def bm_body(blk, carry):
                m = sco_v[pl.ds(blk * 128, 16)]
                for j in range(1, 8):
                    m = jnp.maximum(m, sco_v[pl.ds(blk * 128 + j * 16, 16)])
                plsc.store_scatter(bm_v, [jnp.full((16,), blk, jnp.int32)],
                                   jnp.full((16,), jnp.max(m)), mask=lane0)
                return carry

            lax.fori_loop(0, NBLK, bm_body, 0)
            tail = bm_v[pl.ds(NBPAD - 16, 16)]
            bm_v[pl.ds(NBPAD - 16, 16)] = jnp.where(
                iota + (NBPAD - 16) < NBLK, tail, neg)

            mone = jnp.full((16,), -1.0, jnp.float32)
            for j in range(OUTP // 16):
                outs_v[pl.ds(j * 16, 16)] = mone
                outc_v[pl.ds(j * 16, 16)] = mone
            for j in range(4 * OUTP // 16):
                outb_v[pl.ds(j * 16, 16)] = mone

            def cond(carry):
                go, kept, seen = carry[0], carry[1], carry[2]
                return (go > 0) & (kept < MAX_DET) & (seen < TOTAL_CAND)

            def body(carry):
                go, kept, seen, c0, c1, c2, c3, c4 = carry
                m = neg
                bi = jnp.zeros((16,), jnp.int32)
                for j in range(NBPAD // 16):
                    v = bm_v[pl.ds(j * 16, 16)]
                    upd = v > m
                    m = jnp.where(upd, v, m)
                    bi = jnp.where(upd, iota + j * 16, bi)
                M1 = jnp.max(m)
                blk = jnp.min(jnp.where(m == jnp.full((16,), M1), bi, NBPAD))
                base = blk * 128
                m2 = neg
                pi = jnp.zeros((16,), jnp.int32)
                for j in range(8):
                    v = sco_v[pl.ds(base + j * 16, 16)]
                    upd = v > m2
                    m2 = jnp.where(upd, v, m2)
                    pi = jnp.where(upd, iota + j * 16, pi)
                M = jnp.max(m2)
                Mv = jnp.full((16,), M)
                pos = base + jnp.min(jnp.where(m2 == Mv, pi, NPAD))
                go2 = jnp.sum((m2 > MIN_SCORE).astype(jnp.int32)) > 0

                lvl = ((pos >= BOUNDS[1]).astype(jnp.int32)
                       + (pos >= BOUNDS[2]).astype(jnp.int32)
                       + (pos >= BOUNDS[3]).astype(jnp.int32)
                       + (pos >= BOUNDS[4]).astype(jnp.int32))
                cnt = jnp.where(lvl == 0, c0,
                      jnp.where(lvl == 1, c1,
                      jnp.where(lvl == 2, c2,
                      jnp.where(lvl == 3, c3, c4))))
                cap = jnp.where(lvl == 0, CAPS[0],
                      jnp.where(lvl == 1, CAPS[1],
                      jnp.where(lvl == 2, CAPS[2],
                      jnp.where(lvl == 3, CAPS[3], CAPS[4]))))
                is_cand = (cnt < cap) & go2

                posv = jnp.full((16,), pos, jnp.int32)
                pav = plsc.load_gather(pa_v, [posv])
                pbv = plsc.load_gather(pb_v, [posv])
                cx1 = (pav & 1023).astype(jnp.float32)
                cy1 = ((pav >> 10) & 1023).astype(jnp.float32)
                ccls = ((pav >> 20) & 127).astype(jnp.float32)
                cx2 = (pbv & 1023).astype(jnp.float32)
                cy2 = ((pbv >> 10) & 1023).astype(jnp.float32)
                car = (cx2 - cx1) * (cy2 - cy1)

                sup = jnp.zeros((16,), jnp.bool_)
                keptv = jnp.full((16,), kept)
                for j in range(OUTP // 16):
                    valid = (iota + j * 16) < keptv
                    qx1 = kx1_v[pl.ds(j * 16, 16)]
                    qy1 = ky1_v[pl.ds(j * 16, 16)]
                    qx2 = kx2_v[pl.ds(j * 16, 16)]
                    qy2 = ky2_v[pl.ds(j * 16, 16)]
                    qar = kar_v[pl.ds(j * 16, 16)]
                    xx1 = jnp.maximum(qx1, cx1)
                    yy1 = jnp.maximum(qy1, cy1)
                    xx2 = jnp.minimum(qx2, cx2)
                    yy2 = jnp.minimum(qy2, cy2)
                    inter = (jnp.maximum(xx2 - xx1, 0.0)
                             * jnp.maximum(yy2 - yy1, 0.0))
                    union = qar + car - inter
                    iou = jnp.where(union > 0.0,
                                    inter / jnp.maximum(union, 1e-12), 0.0)
                    sup = sup | (valid & (iou > NMS_TH))
                keep = is_cand & jnp.logical_not(jnp.any(sup))

                @pl.when(go2)
                def _():
                    plsc.store_scatter(sco_v, [posv], neg, mask=lane0)
                    mm = neg
                    for j in range(8):
                        mm = jnp.maximum(mm, sco_v[pl.ds(base + j * 16, 16)])
                    plsc.store_scatter(bm_v, [jnp.full((16,), blk, jnp.int32)],
                                       jnp.full((16,), jnp.max(mm)), mask=lane0)

                @pl.when(keep)
                def _():
                    kidx = jnp.full((16,), kept, jnp.int32)
                    plsc.store_scatter(kx1_v, [kidx], cx1, mask=lane0)
                    plsc.store_scatter(ky1_v, [kidx], cy1, mask=lane0)
                    plsc.store_scatter(kx2_v, [kidx], cx2, mask=lane0)
                    plsc.store_scatter(ky2_v, [kidx], cy2, mask=lane0)
                    plsc.store_scatter(kar_v, [kidx], car, mask=lane0)
                    plsc.store_scatter(outs_v, [kidx], Mv, mask=lane0)
                    plsc.store_scatter(outc_v, [kidx], ccls, mask=lane0)
                    bidx = kidx * 4 + jnp.minimum(iota, 3)
                    bvals = jnp.where(iota == 0, cx1,
                            jnp.where(iota == 1, cy1,
                            jnp.where(iota == 2, cx2, cy2)))
                    plsc.store_scatter(outb_v, [bidx], bvals, mask=iota < 4)

                inc = is_cand.astype(jnp.int32)
                return (go2.astype(jnp.int32),
                        kept + keep.astype(jnp.int32),
                        seen + inc,
                        c0 + jnp.where(lvl == 0, inc, 0),
                        c1 + jnp.where(lvl == 1, inc, 0),
                        c2 + jnp.where(lvl == 2, inc, 0),
                        c3 + jnp.where(lvl == 3, inc, 0),
                        c4 + jnp.where(lvl == 4, inc, 0))

            z = jnp.int32(0)
            lax.while_loop(cond, body, (jnp.int32(1), z, z, z, z, z, z, z))

            pltpu.sync_copy(outs_v, outs_hbm.at[img])
            pltpu.sync_copy(outc_v, outc_hbm.at[img])
            pltpu.sync_copy(outb_v, outb_hbm.at[img])

    return k(scores, pa, pb)


def kernel(cls_head_0, reg_head_0, center_head_0,
           cls_head_1, reg_head_1, center_head_1,
           cls_head_2, reg_head_2, center_head_2,
           cls_head_3, reg_head_3, center_head_3,
           cls_head_4, reg_head_4, center_head_4):
    cls_heads = [cls_head_0, cls_head_1, cls_head_2, cls_head_3, cls_head_4]
    reg_heads = [reg_head_0, reg_head_1, reg_head_2, reg_head_3, reg_head_4]
    ctr_heads = [center_head_0, center_head_1, center_head_2, center_head_3,
                 center_head_4]
    sco_l, pa_l, pb_l = [], [], []
    for li in range(5):
        P = PS[li]
        cls_r = cls_heads[li].reshape(BATCH, P, NUM_CLASSES)
        reg_r = reg_heads[li].reshape(BATCH, P, 4)
        ctr_r = ctr_heads[li].reshape(BATCH, P)
        s_, a_, b_ = _dense_call(li)(cls_r, reg_r, ctr_r)
        sco_l.append(s_)
        pa_l.append(a_)
        pb_l.append(b_)
    S = jnp.concatenate(sco_l, axis=1)
    PA = jnp.concatenate(pa_l, axis=1)
    PB = jnp.concatenate(pb_l, axis=1)
    pad = NPAD - NTOT
    S = jnp.pad(S, ((0, 0), (0, pad)), constant_values=-jnp.inf)
    PA = jnp.pad(PA, ((0, 0), (0, pad)))
    PB = jnp.pad(PB, ((0, 0), (0, pad)))
    outs, outc, outb = _sc_decode(S, PA, PB)
    return (outs[:, :MAX_DET], outc[:, :MAX_DET],
            outb.reshape(BATCH, OUTP, 4)[:, :MAX_DET])


# speedup vs baseline: 283.7732x; 1.0569x over previous
import functools
import numpy as np
import jax
import jax.numpy as jnp
from jax import lax
from jax.experimental import pallas as pl
from jax.experimental.pallas import tpu as pltpu
from jax.experimental.pallas import tpu_sc as plsc

IMAGE_W = 1024
IMAGE_H = 1024
STRIDES = (8, 16, 32, 64, 128)
TOP_N = 1000
MIN_SCORE = 0.05
NMS_TH = 0.6
MAX_DET = 100
NUM_CLASSES = 80
BATCH = 8

PS = tuple((IMAGE_H // s) ** 2 for s in STRIDES)
NTOT = sum(PS)
CH = 512
NSTEP = 43
NPAD = NSTEP * CH
NBLK = NPAD // 128
NBPAD = 176
BOUNDS = tuple(int(x) for x in np.cumsum((0,) + PS))
CAPS = tuple(min(TOP_N, p) for p in PS)
TOTAL_CAND = sum(CAPS)
OUTP = 112


def _decode_chunk(cls, reg, ctr, stride, f, ch, local_i):
    sig = jax.nn.sigmoid(cls)
    ms = jnp.max(sig, axis=2)
    iot = lax.broadcasted_iota(jnp.int32, (BATCH, ch, NUM_CLASSES), 2)
    argm = jnp.min(jnp.where(sig == ms[:, :, None], iot, NUM_CLASSES), axis=2)
    score = jnp.sqrt(ms * jax.nn.sigmoid(ctr))
    regs = jnp.exp(reg)
    p = lax.broadcasted_iota(jnp.int32, (BATCH, ch), 1) + local_i * ch
    a = p // f
    b = p - a * f
    px = (b.astype(jnp.float32) + 0.5) * stride
    py = (a.astype(jnp.float32) + 0.5) * stride
    x1 = jnp.floor(jnp.maximum(px - regs[:, :, 0], 0.0)).astype(jnp.int32)
    y1 = jnp.floor(jnp.maximum(py - regs[:, :, 1], 0.0)).astype(jnp.int32)
    x2 = jnp.minimum(jnp.floor(px + regs[:, :, 2]), IMAGE_W - 1.0).astype(jnp.int32)
    y2 = jnp.minimum(jnp.floor(py + regs[:, :, 3]), IMAGE_H - 1.0).astype(jnp.int32)
    pa = x1 | (y1 << 10) | (argm << 20)
    pb = x2 | (y2 << 10)
    return score, pa, pb


def _dense_body(cls0, reg0, ctr0, cls1, reg1, ctr1, cls2, reg2, ctr2,
                cls3, reg3, ctr3, cls4, reg4, ctr4, sco_ref, pa_ref, pb_ref):
    i = pl.program_id(0)

    def emit(cls_ref, reg_ref, ctr_ref, li, local_i):
        stride = float(STRIDES[li])
        f = IMAGE_H // STRIDES[li]
        s_, a_, b_ = _decode_chunk(cls_ref[...], reg_ref[...], ctr_ref[...],
                                   stride, f, CH, local_i)
        sco_ref[...] = s_
        pa_ref[...] = a_
        pb_ref[...] = b_

    @pl.when(i < 32)
    def _():
        emit(cls0, reg0, ctr0, 0, i)

    @pl.when((i >= 32) & (i < 40))
    def _():
        emit(cls1, reg1, ctr1, 1, i - 32)

    @pl.when((i >= 40) & (i < 42))
    def _():
        emit(cls2, reg2, ctr2, 2, i - 40)

    @pl.when(i == 42)
    def _():
        s3, a3, b3 = _decode_chunk(cls3[...], reg3[...], ctr3[...],
                                   float(STRIDES[3]), IMAGE_H // STRIDES[3],
                                   PS[3], 0)
        s4, a4, b4 = _decode_chunk(cls4[...], reg4[...], ctr4[...],
                                   float(STRIDES[4]), IMAGE_H // STRIDES[4],
                                   PS[4], 0)
        padw = CH - PS[3] - PS[4]
        sco_ref[...] = jnp.concatenate(
            [s3, s4, jnp.full((BATCH, padw), -jnp.inf, jnp.float32)], axis=1)
        pa_ref[...] = jnp.concatenate(
            [a3, a4, jnp.zeros((BATCH, padw), jnp.int32)], axis=1)
        pb_ref[...] = jnp.concatenate(
            [b3, b4, jnp.zeros((BATCH, padw), jnp.int32)], axis=1)


def _dense_call():
    def cspec(P, C, off, hi):
        nch = max(P // CH, 1)
        if C is None:
            return pl.BlockSpec((BATCH, min(P, CH)),
                                lambda i, off=off, hi=hi: (0, jnp.clip(i - off, 0, hi)))
        return pl.BlockSpec((BATCH, min(P, CH), C),
                            lambda i, off=off, hi=hi: (0, jnp.clip(i - off, 0, hi), 0))

    in_specs = []
    offs = (0, 32, 40, 42, 42)
    for li in range(5):
        P = PS[li]
        hi = max(P // CH - 1, 0)
        in_specs.append(cspec(P, NUM_CLASSES, offs[li], hi))
        in_specs.append(cspec(P, 4, offs[li], hi))
        in_specs.append(cspec(P, None, offs[li], hi))

    return pl.pallas_call(
        _dense_body,
        grid=(NSTEP,),
        in_specs=in_specs,
        out_specs=[
            pl.BlockSpec((BATCH, CH), lambda i: (0, i)),
            pl.BlockSpec((BATCH, CH), lambda i: (0, i)),
            pl.BlockSpec((BATCH, CH), lambda i: (0, i)),
        ],
        out_shape=[
            jax.ShapeDtypeStruct((BATCH, NPAD), jnp.float32),
            jax.ShapeDtypeStruct((BATCH, NPAD), jnp.int32),
            jax.ShapeDtypeStruct((BATCH, NPAD), jnp.int32),
        ],
    )


def _sc_decode(scores, pa, pb):
    mesh = plsc.VectorSubcoreMesh(core_axis_name="c", subcore_axis_name="s")

    @functools.partial(
        pl.kernel,
        mesh=mesh,
        compiler_params=pltpu.CompilerParams(needs_layout_passes=False),
        out_type=[
            jax.ShapeDtypeStruct((BATCH, OUTP), jnp.float32),
            jax.ShapeDtypeStruct((BATCH, OUTP), jnp.float32),
            jax.ShapeDtypeStruct((BATCH, 4 * OUTP), jnp.float32),
        ],
        scratch_types=[
            pltpu.VMEM((NPAD,), jnp.float32),
            pltpu.VMEM((NPAD,), jnp.int32),
            pltpu.VMEM((NPAD,), jnp.int32),
            pltpu.VMEM((NBPAD,), jnp.float32),
            pltpu.VMEM((OUTP,), jnp.float32),
            pltpu.VMEM((OUTP,), jnp.float32),
            pltpu.VMEM((OUTP,), jnp.float32),
            pltpu.VMEM((OUTP,), jnp.float32),
            pltpu.VMEM((OUTP,), jnp.float32),
            pltpu.VMEM((OUTP,), jnp.float32),
            pltpu.VMEM((OUTP,), jnp.float32),
            pltpu.VMEM((4 * OUTP,), jnp.float32),
        ],
    )
    def k(sco_hbm, pa_hbm, pb_hbm, outs_hbm, outc_hbm, outb_hbm,
          sco_v, pa_v, pb_v, bm_v, kx1_v, ky1_v, kx2_v, ky2_v, kar_v,
          outs_v, outc_v, outb_v):
        wid = lax.axis_index("s") * 2 + lax.axis_index("c")

        @pl.when(wid < BATCH)
        def _():
            img = wid
            pltpu.sync_copy(sco_hbm.at[img], sco_v)
            pltpu.sync_copy(pa_hbm.at[img], pa_v)
            pltpu.sync_copy(pb_hbm.at[img], pb_v)
            iota = lax.iota(jnp.int32, 16)
            neg = jnp.full((16,), -jnp.inf, jnp.float32)
            lane0 = iota == 0

            def bm_body(blk, carry):
                m = sco_v[pl.ds(blk * 128, 16)]
                for j in range(1, 8):
                    m = jnp.maximum(m, sco_v[pl.ds(blk * 128 + j * 16, 16)])
                plsc.store_scatter(bm_v, [jnp.full((16,), blk, jnp.int32)],
                                   jnp.full((16,), jnp.max(m)), mask=lane0)
                return carry

            lax.fori_loop(0, NBLK, bm_body, 0)
            tail = bm_v[pl.ds(NBPAD - 16, 16)]
            bm_v[pl.ds(NBPAD - 16, 16)] = jnp.where(
                iota + (NBPAD - 16) < NBLK, tail, neg)

            mone = jnp.full((16,), -1.0, jnp.float32)
            for j in range(OUTP // 16):
                outs_v[pl.ds(j * 16, 16)] = mone
                outc_v[pl.ds(j * 16, 16)] = mone
            for j in range(4 * OUTP // 16):
                outb_v[pl.ds(j * 16, 16)] = mone

            def cond(carry):
                go, kept, seen = carry[0], carry[1], carry[2]
                return (go > 0) & (kept < MAX_DET) & (seen < TOTAL_CAND)

            def body(carry):
                go, kept, seen, c0, c1, c2, c3, c4 = carry
                m = neg
                bi = jnp.zeros((16,), jnp.int32)
                for j in range(NBPAD // 16):
                    v = bm_v[pl.ds(j * 16, 16)]
                    upd = v > m
                    m = jnp.where(upd, v, m)
                    bi = jnp.where(upd, iota + j * 16, bi)
                M1 = jnp.max(m)
                blk = jnp.min(jnp.where(m == jnp.full((16,), M1), bi, NBPAD))
                base = blk * 128
                m2 = neg
                pi = jnp.zeros((16,), jnp.int32)
                for j in range(8):
                    v = sco_v[pl.ds(base + j * 16, 16)]
                    upd = v > m2
                    m2 = jnp.where(upd, v, m2)
                    pi = jnp.where(upd, iota + j * 16, pi)
                M = jnp.max(m2)
                Mv = jnp.full((16,), M)
                pos = base + jnp.min(jnp.where(m2 == Mv, pi, NPAD))
                go2 = jnp.sum((m2 > MIN_SCORE).astype(jnp.int32)) > 0

                lvl = ((pos >= BOUNDS[1]).astype(jnp.int32)
                       + (pos >= BOUNDS[2]).astype(jnp.int32)
                       + (pos >= BOUNDS[3]).astype(jnp.int32)
                       + (pos >= BOUNDS[4]).astype(jnp.int32))
                cnt = jnp.where(lvl == 0, c0,
                      jnp.where(lvl == 1, c1,
                      jnp.where(lvl == 2, c2,
                      jnp.where(lvl == 3, c3, c4))))
                cap = jnp.where(lvl == 0, CAPS[0],
                      jnp.where(lvl == 1, CAPS[1],
                      jnp.where(lvl == 2, CAPS[2],
                      jnp.where(lvl == 3, CAPS[3], CAPS[4]))))
                is_cand = (cnt < cap) & go2

                posv = jnp.full((16,), pos, jnp.int32)
                pav = plsc.load_gather(pa_v, [posv])
                pbv = plsc.load_gather(pb_v, [posv])
                cx1 = (pav & 1023).astype(jnp.float32)
                cy1 = ((pav >> 10) & 1023).astype(jnp.float32)
                ccls = ((pav >> 20) & 127).astype(jnp.float32)
                cx2 = (pbv & 1023).astype(jnp.float32)
                cy2 = ((pbv >> 10) & 1023).astype(jnp.float32)
                car = (cx2 - cx1) * (cy2 - cy1)

                sup = jnp.zeros((16,), jnp.bool_)
                keptv = jnp.full((16,), kept)
                for j in range(OUTP // 16):
                    valid = (iota + j * 16) < keptv
                    qx1 = kx1_v[pl.ds(j * 16, 16)]
                    qy1 = ky1_v[pl.ds(j * 16, 16)]
                    qx2 = kx2_v[pl.ds(j * 16, 16)]
                    qy2 = ky2_v[pl.ds(j * 16, 16)]
                    qar = kar_v[pl.ds(j * 16, 16)]
                    xx1 = jnp.maximum(qx1, cx1)
                    yy1 = jnp.maximum(qy1, cy1)
                    xx2 = jnp.minimum(qx2, cx2)
                    yy2 = jnp.minimum(qy2, cy2)
                    inter = (jnp.maximum(xx2 - xx1, 0.0)
                             * jnp.maximum(yy2 - yy1, 0.0))
                    union = qar + car - inter
                    iou = jnp.where(union > 0.0,
                                    inter / jnp.maximum(union, 1e-12), 0.0)
                    sup = sup | (valid & (iou > NMS_TH))
                keep = is_cand & jnp.logical_not(jnp.any(sup))

                @pl.when(go2)
                def _():
                    plsc.store_scatter(sco_v, [posv], neg, mask=lane0)
                    mm = neg
                    for j in range(8):
                        mm = jnp.maximum(mm, sco_v[pl.ds(base + j * 16, 16)])
                    plsc.store_scatter(bm_v, [jnp.full((16,), blk, jnp.int32)],
                                       jnp.full((16,), jnp.max(mm)), mask=lane0)

                @pl.when(keep)
                def _():
                    kidx = jnp.full((16,), kept, jnp.int32)
                    plsc.store_scatter(kx1_v, [kidx], cx1, mask=lane0)
                    plsc.store_scatter(ky1_v, [kidx], cy1, mask=lane0)
                    plsc.store_scatter(kx2_v, [kidx], cx2, mask=lane0)
                    plsc.store_scatter(ky2_v, [kidx], cy2, mask=lane0)
                    plsc.store_scatter(kar_v, [kidx], car, mask=lane0)
                    plsc.store_scatter(outs_v, [kidx], Mv, mask=lane0)
                    plsc.store_scatter(outc_v, [kidx], ccls, mask=lane0)
                    bidx = kidx * 4 + jnp.minimum(iota, 3)
                    bvals = jnp.where(iota == 0, cx1,
                            jnp.where(iota == 1, cy1,
                            jnp.where(iota == 2, cx2, cy2)))
                    plsc.store_scatter(outb_v, [bidx], bvals, mask=iota < 4)

                inc = is_cand.astype(jnp.int32)
                return (go2.astype(jnp.int32),
                        kept + keep.astype(jnp.int32),
                        seen + inc,
                        c0 + jnp.where(lvl == 0, inc, 0),
                        c1 + jnp.where(lvl == 1, inc, 0),
                        c2 + jnp.where(lvl == 2, inc, 0),
                        c3 + jnp.where(lvl == 3, inc, 0),
                        c4 + jnp.where(lvl == 4, inc, 0))

            z = jnp.int32(0)
            lax.while_loop(cond, body, (jnp.int32(1), z, z, z, z, z, z, z))

            pltpu.sync_copy(outs_v, outs_hbm.at[img])
            pltpu.sync_copy(outc_v, outc_hbm.at[img])
            pltpu.sync_copy(outb_v, outb_hbm.at[img])

    return k(scores, pa, pb)


def kernel(cls_head_0, reg_head_0, center_head_0,
           cls_head_1, reg_head_1, center_head_1,
           cls_head_2, reg_head_2, center_head_2,
           cls_head_3, reg_head_3, center_head_3,
           cls_head_4, reg_head_4, center_head_4):
    cls_heads = [cls_head_0, cls_head_1, cls_head_2, cls_head_3, cls_head_4]
    reg_heads = [reg_head_0, reg_head_1, reg_head_2, reg_head_3, reg_head_4]
    ctr_heads = [center_head_0, center_head_1, center_head_2, center_head_3,
                 center_head_4]
    args = []
    for li in range(5):
        P = PS[li]
        args.append(cls_heads[li].reshape(BATCH, P, NUM_CLASSES))
        args.append(reg_heads[li].reshape(BATCH, P, 4))
        args.append(ctr_heads[li].reshape(BATCH, P))
    S, PA, PB = _dense_call()(*args)
    outs, outc, outb = _sc_decode(S, PA, PB)
    return (outs[:, :MAX_DET], outc[:, :MAX_DET],
            outb.reshape(BATCH, OUTP, 4)[:, :MAX_DET])


# speedup vs baseline: 308.6260x; 1.0876x over previous
import functools
import numpy as np
import jax
import jax.numpy as jnp
from jax import lax
from jax.experimental import pallas as pl
from jax.experimental.pallas import tpu as pltpu
from jax.experimental.pallas import tpu_sc as plsc

IMAGE_W = 1024
IMAGE_H = 1024
STRIDES = (8, 16, 32, 64, 128)
TOP_N = 1000
MIN_SCORE = 0.05
NMS_TH = 0.6
MAX_DET = 100
NUM_CLASSES = 80
BATCH = 8

PS = tuple((IMAGE_H // s) ** 2 for s in STRIDES)
NTOT = sum(PS)
CH = 512
NSTEP = 43
NPAD = NSTEP * CH
NBLK = NPAD // 128
NBPAD = 176
BOUNDS = tuple(int(x) for x in np.cumsum((0,) + PS))
CAPS = tuple(min(TOP_N, p) for p in PS)
TOTAL_CAND = sum(CAPS)
OUTP = 112


def _decode_chunk(cls, reg, ctr, stride, f, ch, local_i):
    sig = jax.nn.sigmoid(cls)
    ms = jnp.max(sig, axis=2)
    argm = jnp.argmax(sig, axis=2).astype(jnp.int32)
    score = jnp.sqrt(ms * jax.nn.sigmoid(ctr))
    regs = jnp.exp(reg)
    p = lax.broadcasted_iota(jnp.int32, (BATCH, ch), 1) + local_i * ch
    a = p // f
    b = p - a * f
    px = (b.astype(jnp.float32) + 0.5) * stride
    py = (a.astype(jnp.float32) + 0.5) * stride
    x1 = jnp.floor(jnp.maximum(px - regs[:, :, 0], 0.0)).astype(jnp.int32)
    y1 = jnp.floor(jnp.maximum(py - regs[:, :, 1], 0.0)).astype(jnp.int32)
    x2 = jnp.minimum(jnp.floor(px + regs[:, :, 2]), IMAGE_W - 1.0).astype(jnp.int32)
    y2 = jnp.minimum(jnp.floor(py + regs[:, :, 3]), IMAGE_H - 1.0).astype(jnp.int32)
    pa = x1 | (y1 << 10) | (argm << 20)
    pb = x2 | (y2 << 10)
    return score, pa, pb


def _store_tiles(sco_ref, pa_ref, pb_ref, s_, a_, b_):
    for c in range(CH // 128):
        sco_ref[pl.ds(c * 8, 8), :] = s_[:, c * 128:(c + 1) * 128]
        pa_ref[pl.ds(c * 8, 8), :] = a_[:, c * 128:(c + 1) * 128]
        pb_ref[pl.ds(c * 8, 8), :] = b_[:, c * 128:(c + 1) * 128]


def _dense_body(cls0, reg0, ctr0, cls1, reg1, ctr1, cls2, reg2, ctr2,
                cls3, reg3, ctr3, cls4, reg4, ctr4, sco_ref, pa_ref, pb_ref):
    i = pl.program_id(0)

    def emit(cls_ref, reg_ref, ctr_ref, li, local_i):
        stride = float(STRIDES[li])
        f = IMAGE_H // STRIDES[li]
        s_, a_, b_ = _decode_chunk(cls_ref[...], reg_ref[...], ctr_ref[...],
                                   stride, f, CH, local_i)
        _store_tiles(sco_ref, pa_ref, pb_ref, s_, a_, b_)

    @pl.when(i < 32)
    def _():
        emit(cls0, reg0, ctr0, 0, i)

    @pl.when((i >= 32) & (i < 40))
    def _():
        emit(cls1, reg1, ctr1, 1, i - 32)

    @pl.when((i >= 40) & (i < 42))
    def _():
        emit(cls2, reg2, ctr2, 2, i - 40)

    @pl.when(i == 42)
    def _():
        s3, a3, b3 = _decode_chunk(cls3[...], reg3[...], ctr3[...],
                                   float(STRIDES[3]), IMAGE_H // STRIDES[3],
                                   PS[3], 0)
        s4, a4, b4 = _decode_chunk(cls4[...], reg4[...], ctr4[...],
                                   float(STRIDES[4]), IMAGE_H // STRIDES[4],
                                   PS[4], 0)
        padw = CH - PS[3] - PS[4]
        s_ = jnp.concatenate(
            [s3, s4, jnp.full((BATCH, padw), -jnp.inf, jnp.float32)], axis=1)
        a_ = jnp.concatenate(
            [a3, a4, jnp.zeros((BATCH, padw), jnp.int32)], axis=1)
        b_ = jnp.concatenate(
            [b3, b4, jnp.zeros((BATCH, padw), jnp.int32)], axis=1)
        _store_tiles(sco_ref, pa_ref, pb_ref, s_, a_, b_)


def _dense_call():
    def cspec(P, C, off, hi):
        nch = max(P // CH, 1)
        if C is None:
            return pl.BlockSpec((BATCH, min(P, CH)),
                                lambda i, off=off, hi=hi: (0, jnp.clip(i - off, 0, hi)))
        return pl.BlockSpec((BATCH, min(P, CH), C),
                            lambda i, off=off, hi=hi: (0, jnp.clip(i - off, 0, hi), 0))

    in_specs = []
    offs = (0, 32, 40, 42, 42)
    for li in range(5):
        P = PS[li]
        hi = max(P // CH - 1, 0)
        in_specs.append(cspec(P, NUM_CLASSES, offs[li], hi))
        in_specs.append(cspec(P, 4, offs[li], hi))
        in_specs.append(cspec(P, None, offs[li], hi))

    return pl.pallas_call(
        _dense_body,
        grid=(NSTEP,),
        in_specs=in_specs,
        out_specs=[
            pl.BlockSpec((4 * BATCH, 128), lambda i: (i, 0)),
            pl.BlockSpec((4 * BATCH, 128), lambda i: (i, 0)),
            pl.BlockSpec((4 * BATCH, 128), lambda i: (i, 0)),
        ],
        out_shape=[
            jax.ShapeDtypeStruct((NBLK * BATCH, 128), jnp.float32),
            jax.ShapeDtypeStruct((NBLK * BATCH, 128), jnp.int32),
            jax.ShapeDtypeStruct((NBLK * BATCH, 128), jnp.int32),
        ],
    )


def _sc_decode(scores, pa, pb):
    mesh = plsc.VectorSubcoreMesh(core_axis_name="c", subcore_axis_name="s")

    @functools.partial(
        pl.kernel,
        mesh=mesh,
        compiler_params=pltpu.CompilerParams(needs_layout_passes=False),
        out_type=[
            jax.ShapeDtypeStruct((BATCH, OUTP), jnp.float32),
            jax.ShapeDtypeStruct((BATCH, OUTP), jnp.float32),
            jax.ShapeDtypeStruct((BATCH, 4 * OUTP), jnp.float32),
        ],
        scratch_types=[
            pltpu.VMEM((NBLK, 128), jnp.float32),
            pltpu.VMEM((NBLK, 128), jnp.int32),
            pltpu.VMEM((NBLK, 128), jnp.int32),
            pltpu.VMEM((NBPAD,), jnp.int32),
            pltpu.SemaphoreType.DMA,
            pltpu.VMEM((NBPAD,), jnp.float32),
            pltpu.VMEM((OUTP,), jnp.float32),
            pltpu.VMEM((OUTP,), jnp.float32),
            pltpu.VMEM((OUTP,), jnp.float32),
            pltpu.VMEM((OUTP,), jnp.float32),
            pltpu.VMEM((OUTP,), jnp.float32),
            pltpu.VMEM((OUTP,), jnp.float32),
            pltpu.VMEM((OUTP,), jnp.float32),
            pltpu.VMEM((4 * OUTP,), jnp.float32),
        ],
    )
    def k(sco_hbm, pa_hbm, pb_hbm, outs_hbm, outc_hbm, outb_hbm,
          sco_v, pa_v, pb_v, idx_v, dsem, bm_v, kx1_v, ky1_v, kx2_v, ky2_v,
          kar_v, outs_v, outc_v, outb_v):
        wid = lax.axis_index("s") * 2 + lax.axis_index("c")

        @pl.when(wid < BATCH)
        def _():
            img = wid
            iota = lax.iota(jnp.int32, 16)
            neg = jnp.full((16,), -jnp.inf, jnp.float32)
            lane0 = iota == 0
            for j in range(NBPAD // 16):
                idx_v[pl.ds(j * 16, 16)] = (iota + j * 16) * BATCH + img
            cps = []
            for src, dst in ((sco_hbm, sco_v), (pa_hbm, pa_v), (pb_hbm, pb_v)):
                cps.append(pltpu.async_copy(
                    src.at[idx_v.at[pl.ds(0, 128)]], dst.at[pl.ds(0, 128)],
                    dsem))
                cps.append(pltpu.async_copy(
                    src.at[idx_v.at[pl.ds(128, NBLK - 128)]],
                    dst.at[pl.ds(128, NBLK - 128)], dsem))
            for cp in cps:
                cp.wait()

            def bm_body(blk, carry):
                m = sco_v[blk, pl.ds(0, 16)]
                for j in range(1, 8):
                    m = jnp.maximum(m, sco_v[blk, pl.ds(j * 16, 16)])
                plsc.store_scatter(bm_v, [jnp.full((16,), blk, jnp.int32)],
                                   jnp.full((16,), jnp.max(m)), mask=lane0)
                return carry

            lax.fori_loop(0, NBLK, bm_body, 0)
            tail = bm_v[pl.ds(NBPAD - 16, 16)]
            bm_v[pl.ds(NBPAD - 16, 16)] = jnp.where(
                iota + (NBPAD - 16) < NBLK, tail, neg)

            mone = jnp.full((16,), -1.0, jnp.float32)
            for j in range(OUTP // 16):
                outs_v[pl.ds(j * 16, 16)] = mone
                outc_v[pl.ds(j * 16, 16)] = mone
            for j in range(4 * OUTP // 16):
                outb_v[pl.ds(j * 16, 16)] = mone

            def cond(carry):
                go, kept, seen = carry[0], carry[1], carry[2]
                return (go > 0) & (kept < MAX_DET) & (seen < TOTAL_CAND)

            def body(carry):
                go, kept, seen, c0, c1, c2, c3, c4 = carry
                m = neg
                bi = jnp.zeros((16,), jnp.int32)
                for j in range(NBPAD // 16):
                    v = bm_v[pl.ds(j * 16, 16)]
                    upd = v > m
                    m = jnp.where(upd, v, m)
                    bi = jnp.where(upd, iota + j * 16, bi)
                M1 = jnp.max(m)
                blk = jnp.min(jnp.where(m == jnp.full((16,), M1), bi, NBPAD))
                m2 = neg
                pi = jnp.zeros((16,), jnp.int32)
                for j in range(8):
                    v = sco_v[blk, pl.ds(j * 16, 16)]
                    upd = v > m2
                    m2 = jnp.where(upd, v, m2)
                    pi = jnp.where(upd, iota + j * 16, pi)
                M = jnp.max(m2)
                Mv = jnp.full((16,), M)
                pos_in = jnp.min(jnp.where(m2 == Mv, pi, 128))
                pos = blk * 128 + pos_in
                go2 = jnp.sum((m2 > MIN_SCORE).astype(jnp.int32)) > 0

                lvl = ((pos >= BOUNDS[1]).astype(jnp.int32)
                       + (pos >= BOUNDS[2]).astype(jnp.int32)
                       + (pos >= BOUNDS[3]).astype(jnp.int32)
                       + (pos >= BOUNDS[4]).astype(jnp.int32))
                cnt = jnp.where(lvl == 0, c0,
                      jnp.where(lvl == 1, c1,
                      jnp.where(lvl == 2, c2,
                      jnp.where(lvl == 3, c3, c4))))
                cap = jnp.where(lvl == 0, CAPS[0],
                      jnp.where(lvl == 1, CAPS[1],
                      jnp.where(lvl == 2, CAPS[2],
                      jnp.where(lvl == 3, CAPS[3], CAPS[4]))))
                is_cand = (cnt < cap) & go2

                rowv = jnp.full((16,), blk, jnp.int32)
                colv = jnp.full((16,), pos_in, jnp.int32)
                pav = plsc.load_gather(pa_v, [rowv, colv])
                pbv = plsc.load_gather(pb_v, [rowv, colv])
                cx1 = (pav & 1023).astype(jnp.float32)
                cy1 = ((pav >> 10) & 1023).astype(jnp.float32)
                ccls = ((pav >> 20) & 127).astype(jnp.float32)
                cx2 = (pbv & 1023).astype(jnp.float32)
                cy2 = ((pbv >> 10) & 1023).astype(jnp.float32)
                car = (cx2 - cx1) * (cy2 - cy1)

                sup = jnp.zeros((16,), jnp.bool_)
                keptv = jnp.full((16,), kept)
                for j in range(OUTP // 16):
                    valid = (iota + j * 16) < keptv
                    qx1 = kx1_v[pl.ds(j * 16, 16)]
                    qy1 = ky1_v[pl.ds(j * 16, 16)]
                    qx2 = kx2_v[pl.ds(j * 16, 16)]
                    qy2 = ky2_v[pl.ds(j * 16, 16)]
                    qar = kar_v[pl.ds(j * 16, 16)]
                    xx1 = jnp.maximum(qx1, cx1)
                    yy1 = jnp.maximum(qy1, cy1)
                    xx2 = jnp.minimum(qx2, cx2)
                    yy2 = jnp.minimum(qy2, cy2)
                    inter = (jnp.maximum(xx2 - xx1, 0.0)
                             * jnp.maximum(yy2 - yy1, 0.0))
                    union = qar + car - inter
                    iou = jnp.where(union > 0.0,
                                    inter / jnp.maximum(union, 1e-12), 0.0)
                    sup = sup | (valid & (iou > NMS_TH))
                keep = is_cand & jnp.logical_not(jnp.any(sup))

                @pl.when(go2)
                def _():
                    plsc.store_scatter(sco_v, [rowv, colv], neg, mask=lane0)
                    mm = neg
                    for j in range(8):
                        mm = jnp.maximum(mm, sco_v[blk, pl.ds(j * 16, 16)])
                    plsc.store_scatter(bm_v, [rowv],
                                       jnp.full((16,), jnp.max(mm)), mask=lane0)

                @pl.when(keep)
                def _():
                    kidx = jnp.full((16,), kept, jnp.int32)
                    plsc.store_scatter(kx1_v, [kidx], cx1, mask=lane0)
                    plsc.store_scatter(ky1_v, [kidx], cy1, mask=lane0)
                    plsc.store_scatter(kx2_v, [kidx], cx2, mask=lane0)
                    plsc.store_scatter(ky2_v, [kidx], cy2, mask=lane0)
                    plsc.store_scatter(kar_v, [kidx], car, mask=lane0)
                    plsc.store_scatter(outs_v, [kidx], Mv, mask=lane0)
                    plsc.store_scatter(outc_v, [kidx], ccls, mask=lane0)
                    bidx = kidx * 4 + jnp.minimum(iota, 3)
                    bvals = jnp.where(iota == 0, cx1,
                            jnp.where(iota == 1, cy1,
                            jnp.where(iota == 2, cx2, cy2)))
                    plsc.store_scatter(outb_v, [bidx], bvals, mask=iota < 4)

                inc = is_cand.astype(jnp.int32)
                return (go2.astype(jnp.int32),
                        kept + keep.astype(jnp.int32),
                        seen + inc,
                        c0 + jnp.where(lvl == 0, inc, 0),
                        c1 + jnp.where(lvl == 1, inc, 0),
                        c2 + jnp.where(lvl == 2, inc, 0),
                        c3 + jnp.where(lvl == 3, inc, 0),
                        c4 + jnp.where(lvl == 4, inc, 0))

            z = jnp.int32(0)
            lax.while_loop(cond, body, (jnp.int32(1), z, z, z, z, z, z, z))

            pltpu.sync_copy(outs_v, outs_hbm.at[img])
            pltpu.sync_copy(outc_v, outc_hbm.at[img])
            pltpu.sync_copy(outb_v, outb_hbm.at[img])

    return k(scores, pa, pb)


def kernel(cls_head_0, reg_head_0, center_head_0,
           cls_head_1, reg_head_1, center_head_1,
           cls_head_2, reg_head_2, center_head_2,
           cls_head_3, reg_head_3, center_head_3,
           cls_head_4, reg_head_4, center_head_4):
    cls_heads = [cls_head_0, cls_head_1, cls_head_2, cls_head_3, cls_head_4]
    reg_heads = [reg_head_0, reg_head_1, reg_head_2, reg_head_3, reg_head_4]
    ctr_heads = [center_head_0, center_head_1, center_head_2, center_head_3,
                 center_head_4]
    args = []
    for li in range(5):
        P = PS[li]
        args.append(cls_heads[li].reshape(BATCH, P, NUM_CLASSES))
        args.append(reg_heads[li].reshape(BATCH, P, 4))
        args.append(ctr_heads[li].reshape(BATCH, P))
    S, PA, PB = _dense_call()(*args)
    outs, outc, outb = _sc_decode(S, PA, PB)
    return (outs[:, :MAX_DET], outc[:, :MAX_DET],
            outb.reshape(BATCH, OUTP, 4)[:, :MAX_DET])


# speedup vs baseline: 361.2619x; 1.1705x over previous
import functools
import numpy as np
import jax
import jax.numpy as jnp
from jax import lax
from jax.experimental import pallas as pl
from jax.experimental.pallas import tpu as pltpu
from jax.experimental.pallas import tpu_sc as plsc

IMAGE_W = 1024
IMAGE_H = 1024
STRIDES = (8, 16, 32, 64, 128)
TOP_N = 1000
MIN_SCORE = 0.05
NMS_TH = 0.6
MAX_DET = 100
NUM_CLASSES = 80
BATCH = 8

PS = tuple((IMAGE_H // s) ** 2 for s in STRIDES)
NTOT = sum(PS)
CH = 512
NSTEP = 43
NPAD = NSTEP * CH
NBLK = NPAD // 128
NBPAD = 176
BOUNDS = tuple(int(x) for x in np.cumsum((0,) + PS))
CAPS = tuple(min(TOP_N, p) for p in PS)
TOTAL_CAND = sum(CAPS)
OUTP = 112


def _decode_chunk(cls, reg, ctr, stride, f, ch, local_i):
    sub = min(ch, 128)
    ms_parts, argm_parts = [], []
    for k in range(ch // sub):
        sig = jax.nn.sigmoid(cls[:, k * sub:(k + 1) * sub, :])
        ms_parts.append(jnp.max(sig, axis=2))
        argm_parts.append(jnp.argmax(sig, axis=2).astype(jnp.int32))
    ms = jnp.concatenate(ms_parts, axis=1) if len(ms_parts) > 1 else ms_parts[0]
    argm = (jnp.concatenate(argm_parts, axis=1)
            if len(argm_parts) > 1 else argm_parts[0])
    score = jnp.sqrt(ms * jax.nn.sigmoid(ctr))
    regs = jnp.exp(reg)
    p = lax.broadcasted_iota(jnp.int32, (BATCH, ch), 1) + local_i * ch
    a = p // f
    b = p - a * f
    px = (b.astype(jnp.float32) + 0.5) * stride
    py = (a.astype(jnp.float32) + 0.5) * stride
    x1 = jnp.floor(jnp.maximum(px - regs[:, :, 0], 0.0)).astype(jnp.int32)
    y1 = jnp.floor(jnp.maximum(py - regs[:, :, 1], 0.0)).astype(jnp.int32)
    x2 = jnp.minimum(jnp.floor(px + regs[:, :, 2]), IMAGE_W - 1.0).astype(jnp.int32)
    y2 = jnp.minimum(jnp.floor(py + regs[:, :, 3]), IMAGE_H - 1.0).astype(jnp.int32)
    pa = x1 | (y1 << 10) | (argm << 20)
    pb = x2 | (y2 << 10)
    return score, pa, pb


def _store_tiles(sco_ref, pa_ref, pb_ref, s_, a_, b_):
    for c in range(CH // 128):
        sco_ref[pl.ds(c * 8, 8), :] = s_[:, c * 128:(c + 1) * 128]
        pa_ref[pl.ds(c * 8, 8), :] = a_[:, c * 128:(c + 1) * 128]
        pb_ref[pl.ds(c * 8, 8), :] = b_[:, c * 128:(c + 1) * 128]


def _dense_body(cls0, reg0, ctr0, cls1, reg1, ctr1, cls2, reg2, ctr2,
                cls3, reg3, ctr3, cls4, reg4, ctr4, sco_ref, pa_ref, pb_ref):
    i = pl.program_id(0)

    def emit(cls_ref, reg_ref, ctr_ref, li, local_i):
        stride = float(STRIDES[li])
        f = IMAGE_H // STRIDES[li]
        s_, a_, b_ = _decode_chunk(cls_ref[...], reg_ref[...], ctr_ref[...],
                                   stride, f, CH, local_i)
        _store_tiles(sco_ref, pa_ref, pb_ref, s_, a_, b_)

    @pl.when(i < 32)
    def _():
        emit(cls0, reg0, ctr0, 0, i)

    @pl.when((i >= 32) & (i < 40))
    def _():
        emit(cls1, reg1, ctr1, 1, i - 32)

    @pl.when((i >= 40) & (i < 42))
    def _():
        emit(cls2, reg2, ctr2, 2, i - 40)

    @pl.when(i == 42)
    def _():
        s3, a3, b3 = _decode_chunk(cls3[...], reg3[...], ctr3[...],
                                   float(STRIDES[3]), IMAGE_H // STRIDES[3],
                                   PS[3], 0)
        s4, a4, b4 = _decode_chunk(cls4[...], reg4[...], ctr4[...],
                                   float(STRIDES[4]), IMAGE_H // STRIDES[4],
                                   PS[4], 0)
        padw = CH - PS[3] - PS[4]
        s_ = jnp.concatenate(
            [s3, s4, jnp.full((BATCH, padw), -jnp.inf, jnp.float32)], axis=1)
        a_ = jnp.concatenate(
            [a3, a4, jnp.zeros((BATCH, padw), jnp.int32)], axis=1)
        b_ = jnp.concatenate(
            [b3, b4, jnp.zeros((BATCH, padw), jnp.int32)], axis=1)
        _store_tiles(sco_ref, pa_ref, pb_ref, s_, a_, b_)


def _dense_call():
    def cspec(P, C, off, hi):
        nch = max(P // CH, 1)
        if C is None:
            return pl.BlockSpec((BATCH, min(P, CH)),
                                lambda i, off=off, hi=hi: (0, jnp.clip(i - off, 0, hi)))
        return pl.BlockSpec((BATCH, min(P, CH), C),
                            lambda i, off=off, hi=hi: (0, jnp.clip(i - off, 0, hi), 0))

    in_specs = []
    offs = (0, 32, 40, 42, 42)
    for li in range(5):
        P = PS[li]
        hi = max(P // CH - 1, 0)
        in_specs.append(cspec(P, NUM_CLASSES, offs[li], hi))
        in_specs.append(cspec(P, 4, offs[li], hi))
        in_specs.append(cspec(P, None, offs[li], hi))

    return pl.pallas_call(
        _dense_body,
        grid=(NSTEP,),
        in_specs=in_specs,
        out_specs=[
            pl.BlockSpec((4 * BATCH, 128), lambda i: (i, 0)),
            pl.BlockSpec((4 * BATCH, 128), lambda i: (i, 0)),
            pl.BlockSpec((4 * BATCH, 128), lambda i: (i, 0)),
        ],
        out_shape=[
            jax.ShapeDtypeStruct((NBLK * BATCH, 128), jnp.float32),
            jax.ShapeDtypeStruct((NBLK * BATCH, 128), jnp.int32),
            jax.ShapeDtypeStruct((NBLK * BATCH, 128), jnp.int32),
        ],
    )


def _sc_decode(scores, pa, pb):
    mesh = plsc.VectorSubcoreMesh(core_axis_name="c", subcore_axis_name="s")

    @functools.partial(
        pl.kernel,
        mesh=mesh,
        compiler_params=pltpu.CompilerParams(needs_layout_passes=False),
        out_type=[
            jax.ShapeDtypeStruct((BATCH, OUTP), jnp.float32),
            jax.ShapeDtypeStruct((BATCH, OUTP), jnp.float32),
            jax.ShapeDtypeStruct((BATCH, 4 * OUTP), jnp.float32),
        ],
        scratch_types=[
            pltpu.VMEM((NBLK, 128), jnp.float32),
            pltpu.VMEM((NBLK, 128), jnp.int32),
            pltpu.VMEM((NBLK, 128), jnp.int32),
            pltpu.VMEM((NBPAD,), jnp.int32),
            pltpu.SemaphoreType.DMA,
            pltpu.VMEM((NBPAD,), jnp.float32),
            pltpu.VMEM((OUTP,), jnp.float32),
            pltpu.VMEM((OUTP,), jnp.float32),
            pltpu.VMEM((OUTP,), jnp.float32),
            pltpu.VMEM((OUTP,), jnp.float32),
            pltpu.VMEM((OUTP,), jnp.float32),
            pltpu.VMEM((OUTP,), jnp.float32),
            pltpu.VMEM((OUTP,), jnp.float32),
            pltpu.VMEM((4 * OUTP,), jnp.float32),
        ],
    )
    def k(sco_hbm, pa_hbm, pb_hbm, outs_hbm, outc_hbm, outb_hbm,
          sco_v, pa_v, pb_v, idx_v, dsem, bm_v, kx1_v, ky1_v, kx2_v, ky2_v,
          kar_v, outs_v, outc_v, outb_v):
        wid = lax.axis_index("s") * 2 + lax.axis_index("c")

        @pl.when(wid < BATCH)
        def _():
            img = wid
            iota = lax.iota(jnp.int32, 16)
            neg = jnp.full((16,), -jnp.inf, jnp.float32)
            lane0 = iota == 0
            for j in range(NBPAD // 16):
                idx_v[pl.ds(j * 16, 16)] = (iota + j * 16) * BATCH + img
            cps = []
            for src, dst in ((sco_hbm, sco_v), (pa_hbm, pa_v), (pb_hbm, pb_v)):
                cps.append(pltpu.async_copy(
                    src.at[idx_v.at[pl.ds(0, 128)]], dst.at[pl.ds(0, 128)],
                    dsem))
                cps.append(pltpu.async_copy(
                    src.at[idx_v.at[pl.ds(128, NBLK - 128)]],
                    dst.at[pl.ds(128, NBLK - 128)], dsem))
            for cp in cps:
                cp.wait()

            def bm_body(blk, carry):
                m = sco_v[blk, pl.ds(0, 16)]
                for j in range(1, 8):
                    m = jnp.maximum(m, sco_v[blk, pl.ds(j * 16, 16)])
                plsc.store_scatter(bm_v, [jnp.full((16,), blk, jnp.int32)],
                                   jnp.full((16,), jnp.max(m)), mask=lane0)
                return carry

            lax.fori_loop(0, NBLK, bm_body, 0)
            tail = bm_v[pl.ds(NBPAD - 16, 16)]
            bm_v[pl.ds(NBPAD - 16, 16)] = jnp.where(
                iota + (NBPAD - 16) < NBLK, tail, neg)

            mone = jnp.full((16,), -1.0, jnp.float32)
            for j in range(OUTP // 16):
                outs_v[pl.ds(j * 16, 16)] = mone
                outc_v[pl.ds(j * 16, 16)] = mone
            for j in range(4 * OUTP // 16):
                outb_v[pl.ds(j * 16, 16)] = mone

            def cond(carry):
                go, kept, seen = carry[0], carry[1], carry[2]
                return (go > 0) & (kept < MAX_DET) & (seen < TOTAL_CAND)

            def body(carry):
                go, kept, seen, c0, c1, c2, c3, c4 = carry
                m = neg
                bi = jnp.zeros((16,), jnp.int32)
                for j in range(NBPAD // 16):
                    v = bm_v[pl.ds(j * 16, 16)]
                    upd = v > m
                    m = jnp.where(upd, v, m)
                    bi = jnp.where(upd, iota + j * 16, bi)
                M1 = jnp.max(m)
                blk = jnp.min(jnp.where(m == jnp.full((16,), M1), bi, NBPAD))
                m2 = neg
                pi = jnp.zeros((16,), jnp.int32)
                for j in range(8):
                    v = sco_v[blk, pl.ds(j * 16, 16)]
                    upd = v > m2
                    m2 = jnp.where(upd, v, m2)
                    pi = jnp.where(upd, iota + j * 16, pi)
                M = jnp.max(m2)
                Mv = jnp.full((16,), M)
                pos_in = jnp.min(jnp.where(m2 == Mv, pi, 128))
                pos = blk * 128 + pos_in
                go2 = jnp.sum((m2 > MIN_SCORE).astype(jnp.int32)) > 0

                lvl = ((pos >= BOUNDS[1]).astype(jnp.int32)
                       + (pos >= BOUNDS[2]).astype(jnp.int32)
                       + (pos >= BOUNDS[3]).astype(jnp.int32)
                       + (pos >= BOUNDS[4]).astype(jnp.int32))
                cnt = jnp.where(lvl == 0, c0,
                      jnp.where(lvl == 1, c1,
                      jnp.where(lvl == 2, c2,
                      jnp.where(lvl == 3, c3, c4))))
                cap = jnp.where(lvl == 0, CAPS[0],
                      jnp.where(lvl == 1, CAPS[1],
                      jnp.where(lvl == 2, CAPS[2],
                      jnp.where(lvl == 3, CAPS[3], CAPS[4]))))
                is_cand = (cnt < cap) & go2

                rowv = jnp.full((16,), blk, jnp.int32)
                colv = jnp.full((16,), pos_in, jnp.int32)
                pav = plsc.load_gather(pa_v, [rowv, colv])
                pbv = plsc.load_gather(pb_v, [rowv, colv])
                cx1 = (pav & 1023).astype(jnp.float32)
                cy1 = ((pav >> 10) & 1023).astype(jnp.float32)
                ccls = ((pav >> 20) & 127).astype(jnp.float32)
                cx2 = (pbv & 1023).astype(jnp.float32)
                cy2 = ((pbv >> 10) & 1023).astype(jnp.float32)
                car = (cx2 - cx1) * (cy2 - cy1)

                sup = jnp.zeros((16,), jnp.bool_)
                keptv = jnp.full((16,), kept)
                for j in range(OUTP // 16):
                    valid = (iota + j * 16) < keptv
                    qx1 = kx1_v[pl.ds(j * 16, 16)]
                    qy1 = ky1_v[pl.ds(j * 16, 16)]
                    qx2 = kx2_v[pl.ds(j * 16, 16)]
                    qy2 = ky2_v[pl.ds(j * 16, 16)]
                    qar = kar_v[pl.ds(j * 16, 16)]
                    xx1 = jnp.maximum(qx1, cx1)
                    yy1 = jnp.maximum(qy1, cy1)
                    xx2 = jnp.minimum(qx2, cx2)
                    yy2 = jnp.minimum(qy2, cy2)
                    inter = (jnp.maximum(xx2 - xx1, 0.0)
                             * jnp.maximum(yy2 - yy1, 0.0))
                    union = qar + car - inter
                    iou = jnp.where(union > 0.0,
                                    inter / jnp.maximum(union, 1e-12), 0.0)
                    sup = sup | (valid & (iou > NMS_TH))
                keep = is_cand & jnp.logical_not(jnp.any(sup))

                @pl.when(go2)
                def _():
                    plsc.store_scatter(sco_v, [rowv, colv], neg, mask=lane0)
                    mm = neg
                    for j in range(8):
                        mm = jnp.maximum(mm, sco_v[blk, pl.ds(j * 16, 16)])
                    plsc.store_scatter(bm_v, [rowv],
                                       jnp.full((16,), jnp.max(mm)), mask=lane0)

                @pl.when(keep)
                def _():
                    kidx = jnp.full((16,), kept, jnp.int32)
                    plsc.store_scatter(kx1_v, [kidx], cx1, mask=lane0)
                    plsc.store_scatter(ky1_v, [kidx], cy1, mask=lane0)
                    plsc.store_scatter(kx2_v, [kidx], cx2, mask=lane0)
                    plsc.store_scatter(ky2_v, [kidx], cy2, mask=lane0)
                    plsc.store_scatter(kar_v, [kidx], car, mask=lane0)
                    plsc.store_scatter(outs_v, [kidx], Mv, mask=lane0)
                    plsc.store_scatter(outc_v, [kidx], ccls, mask=lane0)
                    bidx = kidx * 4 + jnp.minimum(iota, 3)
                    bvals = jnp.where(iota == 0, cx1,
                            jnp.where(iota == 1, cy1,
                            jnp.where(iota == 2, cx2, cy2)))
                    plsc.store_scatter(outb_v, [bidx], bvals, mask=iota < 4)

                inc = is_cand.astype(jnp.int32)
                return (go2.astype(jnp.int32),
                        kept + keep.astype(jnp.int32),
                        seen + inc,
                        c0 + jnp.where(lvl == 0, inc, 0),
                        c1 + jnp.where(lvl == 1, inc, 0),
                        c2 + jnp.where(lvl == 2, inc, 0),
                        c3 + jnp.where(lvl == 3, inc, 0),
                        c4 + jnp.where(lvl == 4, inc, 0))

            z = jnp.int32(0)
            lax.while_loop(cond, body, (jnp.int32(1), z, z, z, z, z, z, z))

            pltpu.sync_copy(outs_v, outs_hbm.at[img])
            pltpu.sync_copy(outc_v, outc_hbm.at[img])
            pltpu.sync_copy(outb_v, outb_hbm.at[img])

    return k(scores, pa, pb)


def kernel(cls_head_0, reg_head_0, center_head_0,
           cls_head_1, reg_head_1, center_head_1,
           cls_head_2, reg_head_2, center_head_2,
           cls_head_3, reg_head_3, center_head_3,
           cls_head_4, reg_head_4, center_head_4):
    cls_heads = [cls_head_0, cls_head_1, cls_head_2, cls_head_3, cls_head_4]
    reg_heads = [reg_head_0, reg_head_1, reg_head_2, reg_head_3, reg_head_4]
    ctr_heads = [center_head_0, center_head_1, center_head_2, center_head_3,
                 center_head_4]
    args = []
    for li in range(5):
        P = PS[li]
        args.append(cls_heads[li].reshape(BATCH, P, NUM_CLASSES))
        args.append(reg_heads[li].reshape(BATCH, P, 4))
        args.append(ctr_heads[li].reshape(BATCH, P))
    S, PA, PB = _dense_call()(*args)
    outs, outc, outb = _sc_decode(S, PA, PB)
    return (outs[:, :MAX_DET], outc[:, :MAX_DET],
            outb.reshape(BATCH, OUTP, 4)[:, :MAX_DET])


# speedup vs baseline: 362.6680x; 1.0039x over previous
import functools
import numpy as np
import jax
import jax.numpy as jnp
from jax import lax
from jax.experimental import pallas as pl
from jax.experimental.pallas import tpu as pltpu
from jax.experimental.pallas import tpu_sc as plsc

IMAGE_W = 1024
IMAGE_H = 1024
STRIDES = (8, 16, 32, 64, 128)
TOP_N = 1000
MIN_SCORE = 0.05
NMS_TH = 0.6
MAX_DET = 100
NUM_CLASSES = 80
BATCH = 8

PS = tuple((IMAGE_H // s) ** 2 for s in STRIDES)
NTOT = sum(PS)
CH = 512
NSTEP = 43
NPAD = NSTEP * CH
NBLK = NPAD // 128
NBPAD = 176
BOUNDS = tuple(int(x) for x in np.cumsum((0,) + PS))
CAPS = tuple(min(TOP_N, p) for p in PS)
TOTAL_CAND = sum(CAPS)
OUTP = 112


def _decode_chunk(cls, reg, ctr, stride, f, ch, local_i):
    sub = min(ch, 128)
    ms_parts, argm_parts = [], []
    for k in range(ch // sub):
        sig = jax.nn.sigmoid(cls[:, k * sub:(k + 1) * sub, :])
        ms_parts.append(jnp.max(sig, axis=2))
        argm_parts.append(jnp.argmax(sig, axis=2).astype(jnp.int32))
    ms = jnp.concatenate(ms_parts, axis=1) if len(ms_parts) > 1 else ms_parts[0]
    argm = (jnp.concatenate(argm_parts, axis=1)
            if len(argm_parts) > 1 else argm_parts[0])
    score = jnp.sqrt(ms * jax.nn.sigmoid(ctr))
    regs = jnp.exp(reg)
    p = lax.broadcasted_iota(jnp.int32, (BATCH, ch), 1) + local_i * ch
    a = p // f
    b = p - a * f
    px = (b.astype(jnp.float32) + 0.5) * stride
    py = (a.astype(jnp.float32) + 0.5) * stride
    x1 = jnp.floor(jnp.maximum(px - regs[:, :, 0], 0.0)).astype(jnp.int32)
    y1 = jnp.floor(jnp.maximum(py - regs[:, :, 1], 0.0)).astype(jnp.int32)
    x2 = jnp.minimum(jnp.floor(px + regs[:, :, 2]), IMAGE_W - 1.0).astype(jnp.int32)
    y2 = jnp.minimum(jnp.floor(py + regs[:, :, 3]), IMAGE_H - 1.0).astype(jnp.int32)
    pa = x1 | (y1 << 10) | (argm << 20)
    pb = x2 | (y2 << 10)
    return score, pa, pb


def _store_tiles(sco_ref, pa_ref, pb_ref, s_, a_, b_):
    for c in range(CH // 128):
        sco_ref[pl.ds(c * 8, 8), :] = s_[:, c * 128:(c + 1) * 128]
        pa_ref[pl.ds(c * 8, 8), :] = a_[:, c * 128:(c + 1) * 128]
        pb_ref[pl.ds(c * 8, 8), :] = b_[:, c * 128:(c + 1) * 128]


def _dense_body(cls0, reg0, ctr0, cls1, reg1, ctr1, cls2, reg2, ctr2,
                cls3, reg3, ctr3, cls4, reg4, ctr4, sco_ref, pa_ref, pb_ref):
    i = pl.program_id(0)

    def emit(cls_ref, reg_ref, ctr_ref, li, local_i):
        stride = float(STRIDES[li])
        f = IMAGE_H // STRIDES[li]
        s_, a_, b_ = _decode_chunk(cls_ref[...], reg_ref[...], ctr_ref[...],
                                   stride, f, CH, local_i)
        _store_tiles(sco_ref, pa_ref, pb_ref, s_, a_, b_)

    @pl.when(i < 32)
    def _():
        emit(cls0, reg0, ctr0, 0, i)

    @pl.when((i >= 32) & (i < 40))
    def _():
        emit(cls1, reg1, ctr1, 1, i - 32)

    @pl.when((i >= 40) & (i < 42))
    def _():
        emit(cls2, reg2, ctr2, 2, i - 40)

    @pl.when(i == 42)
    def _():
        s3, a3, b3 = _decode_chunk(cls3[...], reg3[...], ctr3[...],
                                   float(STRIDES[3]), IMAGE_H // STRIDES[3],
                                   PS[3], 0)
        s4, a4, b4 = _decode_chunk(cls4[...], reg4[...], ctr4[...],
                                   float(STRIDES[4]), IMAGE_H // STRIDES[4],
                                   PS[4], 0)
        padw = CH - PS[3] - PS[4]
        s_ = jnp.concatenate(
            [s3, s4, jnp.full((BATCH, padw), -jnp.inf, jnp.float32)], axis=1)
        a_ = jnp.concatenate(
            [a3, a4, jnp.zeros((BATCH, padw), jnp.int32)], axis=1)
        b_ = jnp.concatenate(
            [b3, b4, jnp.zeros((BATCH, padw), jnp.int32)], axis=1)
        _store_tiles(sco_ref, pa_ref, pb_ref, s_, a_, b_)


def _dense_call():
    def cspec(P, C, off, hi):
        nch = max(P // CH, 1)
        if C is None:
            return pl.BlockSpec((BATCH, min(P, CH)),
                                lambda i, off=off, hi=hi: (0, jnp.clip(i - off, 0, hi)))
        return pl.BlockSpec((BATCH, min(P, CH), C),
                            lambda i, off=off, hi=hi: (0, jnp.clip(i - off, 0, hi), 0))

    in_specs = []
    offs = (0, 32, 40, 42, 42)
    for li in range(5):
        P = PS[li]
        hi = max(P // CH - 1, 0)
        in_specs.append(cspec(P, NUM_CLASSES, offs[li], hi))
        in_specs.append(cspec(P, 4, offs[li], hi))
        in_specs.append(cspec(P, None, offs[li], hi))

    return pl.pallas_call(
        _dense_body,
        grid=(NSTEP,),
        in_specs=in_specs,
        out_specs=[
            pl.BlockSpec((4 * BATCH, 128), lambda i: (i, 0)),
            pl.BlockSpec((4 * BATCH, 128), lambda i: (i, 0)),
            pl.BlockSpec((4 * BATCH, 128), lambda i: (i, 0)),
        ],
        out_shape=[
            jax.ShapeDtypeStruct((NBLK * BATCH, 128), jnp.float32),
            jax.ShapeDtypeStruct((NBLK * BATCH, 128), jnp.int32),
            jax.ShapeDtypeStruct((NBLK * BATCH, 128), jnp.int32),
        ],
    )


def _sc_decode(scores, pa, pb):
    mesh = plsc.VectorSubcoreMesh(core_axis_name="c", subcore_axis_name="s")

    @functools.partial(
        pl.kernel,
        mesh=mesh,
        compiler_params=pltpu.CompilerParams(needs_layout_passes=False, use_tc_tiling_on_sc=True),
        out_type=[
            jax.ShapeDtypeStruct((BATCH, OUTP), jnp.float32),
            jax.ShapeDtypeStruct((BATCH, OUTP), jnp.float32),
            jax.ShapeDtypeStruct((BATCH, 4 * OUTP), jnp.float32),
        ],
        scratch_types=[
            pltpu.VMEM((NBLK, 128), jnp.float32),
            pltpu.VMEM((NBLK, 128), jnp.int32),
            pltpu.VMEM((NBLK, 128), jnp.int32),
            pltpu.VMEM((NBPAD,), jnp.int32),
            pltpu.SemaphoreType.DMA,
            pltpu.VMEM((NBPAD,), jnp.float32),
            pltpu.VMEM((OUTP,), jnp.float32),
            pltpu.VMEM((OUTP,), jnp.float32),
            pltpu.VMEM((OUTP,), jnp.float32),
            pltpu.VMEM((OUTP,), jnp.float32),
            pltpu.VMEM((OUTP,), jnp.float32),
            pltpu.VMEM((OUTP,), jnp.float32),
            pltpu.VMEM((OUTP,), jnp.float32),
            pltpu.VMEM((4 * OUTP,), jnp.float32),
        ],
    )
    def k(sco_hbm, pa_hbm, pb_hbm, outs_hbm, outc_hbm, outb_hbm,
          sco_v, pa_v, pb_v, idx_v, dsem, bm_v, kx1_v, ky1_v, kx2_v, ky2_v,
          kar_v, outs_v, outc_v, outb_v):
        wid = lax.axis_index("s") * 2 + lax.axis_index("c")

        @pl.when(wid < BATCH)
        def _():
            img = wid
            iota = lax.iota(jnp.int32, 16)
            neg = jnp.full((16,), -jnp.inf, jnp.float32)
            lane0 = iota == 0
            for j in range(NBPAD // 16):
                idx_v[pl.ds(j * 16, 16)] = (iota + j * 16) * BATCH + img
            cps = []
            for src, dst in ((sco_hbm, sco_v), (pa_hbm, pa_v), (pb_hbm, pb_v)):
                cps.append(pltpu.async_copy(
                    src.at[idx_v.at[pl.ds(0, 128)]], dst.at[pl.ds(0, 128)],
                    dsem))
                cps.append(pltpu.async_copy(
                    src.at[idx_v.at[pl.ds(128, NBLK - 128)]],
                    dst.at[pl.ds(128, NBLK - 128)], dsem))
            for cp in cps:
                cp.wait()

            def bm_body(blk, carry):
                m = sco_v[blk, pl.ds(0, 16)]
                for j in range(1, 8):
                    m = jnp.maximum(m, sco_v[blk, pl.ds(j * 16, 16)])
                plsc.store_scatter(bm_v, [jnp.full((16,), blk, jnp.int32)],
                                   jnp.full((16,), jnp.max(m)), mask=lane0)
                return carry

            lax.fori_loop(0, NBLK, bm_body, 0)
            tail = bm_v[pl.ds(NBPAD - 16, 16)]
            bm_v[pl.ds(NBPAD - 16, 16)] = jnp.where(
                iota + (NBPAD - 16) < NBLK, tail, neg)

            mone = jnp.full((16,), -1.0, jnp.float32)
            for j in range(OUTP // 16):
                outs_v[pl.ds(j * 16, 16)] = mone
                outc_v[pl.ds(j * 16, 16)] = mone
            for j in range(4 * OUTP // 16):
                outb_v[pl.ds(j * 16, 16)] = mone

            def cond(carry):
                go, kept, seen = carry[0], carry[1], carry[2]
                return (go > 0) & (kept < MAX_DET) & (seen < TOTAL_CAND)

            def body(carry):
                go, kept, seen, c0, c1, c2, c3, c4 = carry
                m = neg
                bi = jnp.zeros((16,), jnp.int32)
                for j in range(NBPAD // 16):
                    v = bm_v[pl.ds(j * 16, 16)]
                    upd = v > m
                    m = jnp.where(upd, v, m)
                    bi = jnp.where(upd, iota + j * 16, bi)
                M1 = jnp.max(m)
                blk = jnp.min(jnp.where(m == jnp.full((16,), M1), bi, NBPAD))
                m2 = neg
                pi = jnp.zeros((16,), jnp.int32)
                for j in range(8):
                    v = sco_v[blk, pl.ds(j * 16, 16)]
                    upd = v > m2
                    m2 = jnp.where(upd, v, m2)
                    pi = jnp.where(upd, iota + j * 16, pi)
                M = jnp.max(m2)
                Mv = jnp.full((16,), M)
                pos_in = jnp.min(jnp.where(m2 == Mv, pi, 128))
                pos = blk * 128 + pos_in
                go2 = jnp.sum((m2 > MIN_SCORE).astype(jnp.int32)) > 0

                lvl = ((pos >= BOUNDS[1]).astype(jnp.int32)
                       + (pos >= BOUNDS[2]).astype(jnp.int32)
                       + (pos >= BOUNDS[3]).astype(jnp.int32)
                       + (pos >= BOUNDS[4]).astype(jnp.int32))
                cnt = jnp.where(lvl == 0, c0,
                      jnp.where(lvl == 1, c1,
                      jnp.where(lvl == 2, c2,
                      jnp.where(lvl == 3, c3, c4))))
                cap = jnp.where(lvl == 0, CAPS[0],
                      jnp.where(lvl == 1, CAPS[1],
                      jnp.where(lvl == 2, CAPS[2],
                      jnp.where(lvl == 3, CAPS[3], CAPS[4]))))
                is_cand = (cnt < cap) & go2

                rowv = jnp.full((16,), blk, jnp.int32)
                colv = jnp.full((16,), pos_in, jnp.int32)
                pav = plsc.load_gather(pa_v, [rowv, colv])
                pbv = plsc.load_gather(pb_v, [rowv, colv])
                cx1 = (pav & 1023).astype(jnp.float32)
                cy1 = ((pav >> 10) & 1023).astype(jnp.float32)
                ccls = ((pav >> 20) & 127).astype(jnp.float32)
                cx2 = (pbv & 1023).astype(jnp.float32)
                cy2 = ((pbv >> 10) & 1023).astype(jnp.float32)
                car = (cx2 - cx1) * (cy2 - cy1)

                sup = jnp.zeros((16,), jnp.bool_)
                keptv = jnp.full((16,), kept)
                for j in range(OUTP // 16):
                    valid = (iota + j * 16) < keptv
                    qx1 = kx1_v[pl.ds(j * 16, 16)]
                    qy1 = ky1_v[pl.ds(j * 16, 16)]
                    qx2 = kx2_v[pl.ds(j * 16, 16)]
                    qy2 = ky2_v[pl.ds(j * 16, 16)]
                    qar = kar_v[pl.ds(j * 16, 16)]
                    xx1 = jnp.maximum(qx1, cx1)
                    yy1 = jnp.maximum(qy1, cy1)
                    xx2 = jnp.minimum(qx2, cx2)
                    yy2 = jnp.minimum(qy2, cy2)
                    inter = (jnp.maximum(xx2 - xx1, 0.0)
                             * jnp.maximum(yy2 - yy1, 0.0))
                    union = qar + car - inter
                    iou = jnp.where(union > 0.0,
                                    inter / jnp.maximum(union, 1e-12), 0.0)
                    sup = sup | (valid & (iou > NMS_TH))
                keep = is_cand & jnp.logical_not(jnp.any(sup))

                @pl.when(go2)
                def _():
                    plsc.store_scatter(sco_v, [rowv, colv], neg, mask=lane0)
                    mm = neg
                    for j in range(8):
                        mm = jnp.maximum(mm, sco_v[blk, pl.ds(j * 16, 16)])
                    plsc.store_scatter(bm_v, [rowv],
                                       jnp.full((16,), jnp.max(mm)), mask=lane0)

                @pl.when(keep)
                def _():
                    kidx = jnp.full((16,), kept, jnp.int32)
                    plsc.store_scatter(kx1_v, [kidx], cx1, mask=lane0)
                    plsc.store_scatter(ky1_v, [kidx], cy1, mask=lane0)
                    plsc.store_scatter(kx2_v, [kidx], cx2, mask=lane0)
                    plsc.store_scatter(ky2_v, [kidx], cy2, mask=lane0)
                    plsc.store_scatter(kar_v, [kidx], car, mask=lane0)
                    plsc.store_scatter(outs_v, [kidx], Mv, mask=lane0)
                    plsc.store_scatter(outc_v, [kidx], ccls, mask=lane0)
                    bidx = kidx * 4 + jnp.minimum(iota, 3)
                    bvals = jnp.where(iota == 0, cx1,
                            jnp.where(iota == 1, cy1,
                            jnp.where(iota == 2, cx2, cy2)))
                    plsc.store_scatter(outb_v, [bidx], bvals, mask=iota < 4)

                inc = is_cand.astype(jnp.int32)
                return (go2.astype(jnp.int32),
                        kept + keep.astype(jnp.int32),
                        seen + inc,
                        c0 + jnp.where(lvl == 0, inc, 0),
                        c1 + jnp.where(lvl == 1, inc, 0),
                        c2 + jnp.where(lvl == 2, inc, 0),
                        c3 + jnp.where(lvl == 3, inc, 0),
                        c4 + jnp.where(lvl == 4, inc, 0))

            z = jnp.int32(0)
            lax.while_loop(cond, body, (jnp.int32(1), z, z, z, z, z, z, z))

            pltpu.sync_copy(outs_v, outs_hbm.at[img])
            pltpu.sync_copy(outc_v, outc_hbm.at[img])
            pltpu.sync_copy(outb_v, outb_hbm.at[img])

    return k(scores, pa, pb)


def kernel(cls_head_0, reg_head_0, center_head_0,
           cls_head_1, reg_head_1, center_head_1,
           cls_head_2, reg_head_2, center_head_2,
           cls_head_3, reg_head_3, center_head_3,
           cls_head_4, reg_head_4, center_head_4):
    cls_heads = [cls_head_0, cls_head_1, cls_head_2, cls_head_3, cls_head_4]
    reg_heads = [reg_head_0, reg_head_1, reg_head_2, reg_head_3, reg_head_4]
    ctr_heads = [center_head_0, center_head_1, center_head_2, center_head_3,
                 center_head_4]
    args = []
    for li in range(5):
        P = PS[li]
        args.append(cls_heads[li].reshape(BATCH, P, NUM_CLASSES))
        args.append(reg_heads[li].reshape(BATCH, P, 4))
        args.append(ctr_heads[li].reshape(BATCH, P))
    S, PA, PB = _dense_call()(*args)
    outs, outc, outb = _sc_decode(S, PA, PB)
    return (outs[:, :MAX_DET], outc[:, :MAX_DET],
            outb.reshape(BATCH, OUTP, 4)[:, :MAX_DET])
